# baseline (speedup 1.0000x reference)
import functools

import jax
import jax.numpy as jnp
from jax.experimental import pallas as pl
from jax.experimental.pallas import tpu as pltpu

_FC = 8
_NW = 6
_NO = 4


def _experts_kernel(x_hbm, w_hbm, b_vmem, o_hbm,
                    xb, wb, ob, xsem, wsem, osem):
    B, E, N, D = x_hbm.shape
    BFC = D // _FC
    G = E * _FC

    def w_copy(g):
        e, fc = divmod(g, _FC)
        return pltpu.make_async_copy(
            w_hbm.at[e, pl.ds(fc * BFC, BFC), :], wb.at[g % _NW],
            wsem.at[g % _NW])

    def x_copy(e):
        return pltpu.make_async_copy(
            x_hbm.at[:, e], xb.at[e % 2], xsem.at[e % 2])

    def o_copy(g):
        e, fc = divmod(g, _FC)
        return pltpu.make_async_copy(
            ob.at[g % _NO], o_hbm.at[:, e, :, pl.ds(fc * BFC, BFC)],
            osem.at[g % _NO])

    x_copy(0).start()
    for g in range(min(_NW, G)):
        w_copy(g).start()

    for g in range(G):
        e, fc = divmod(g, _FC)
        if fc == 0:
            x_copy(e).wait()
        w_copy(g).wait()
        if g >= _NO:
            o_copy(g - _NO).wait()
        bias_row = b_vmem[e, 0, fc * BFC:(fc + 1) * BFC]
        for bb in range(B):
            acc = jax.lax.dot_general(
                xb[e % 2, bb], wb[g % _NW],
                dimension_numbers=(((1,), (1,)), ((), ())),
                preferred_element_type=jnp.float32,
            )
            ob[g % _NO, bb] = acc + bias_row[None, :]
        o_copy(g).start()
        if g + _NW < G:
            w_copy(g + _NW).start()
        if fc == 0 and e + 1 < E:
            x_copy(e + 1).start()

    for g in range(max(G - _NO, 0), G):
        o_copy(g).wait()


@jax.jit
def kernel(x, W, b):
    B, E, N, D = x.shape
    BFC = D // _FC
    b3 = b.reshape(E, 1, D)
    return pl.pallas_call(
        _experts_kernel,
        in_specs=[
            pl.BlockSpec(memory_space=pltpu.HBM),
            pl.BlockSpec(memory_space=pltpu.HBM),
            pl.BlockSpec(memory_space=pltpu.VMEM),
        ],
        out_specs=pl.BlockSpec(memory_space=pltpu.HBM),
        out_shape=jax.ShapeDtypeStruct((B, E, N, D), x.dtype),
        scratch_shapes=[
            pltpu.VMEM((2, B, N, D), jnp.float32),
            pltpu.VMEM((_NW, BFC, D), jnp.float32),
            pltpu.VMEM((_NO, B, N, BFC), jnp.float32),
            pltpu.SemaphoreType.DMA((2,)),
            pltpu.SemaphoreType.DMA((_NW,)),
            pltpu.SemaphoreType.DMA((_NO,)),
        ],
        compiler_params=pltpu.CompilerParams(
            vmem_limit_bytes=100 * 1024 * 1024),
    )(x, W, b3)


# speedup vs baseline: 1.0769x; 1.0769x over previous
import functools

import jax
import jax.numpy as jnp
from jax.experimental import pallas as pl
from jax.experimental.pallas import tpu as pltpu

_FC = 4
_NW = 4
_NO = 4


def _experts_kernel(x_hbm, w_hbm, b_vmem, o_hbm,
                    xb, wb, ob, xsem, wsem, osem):
    B, E, N, D = x_hbm.shape
    BFC = D // _FC
    G = E * _FC

    def w_copy(g):
        e, fc = divmod(g, _FC)
        return pltpu.make_async_copy(
            w_hbm.at[e, pl.ds(fc * BFC, BFC), :], wb.at[g % _NW],
            wsem.at[g % _NW])

    def x_copy(e):
        return pltpu.make_async_copy(
            x_hbm.at[:, e], xb.at[e % 2], xsem.at[e % 2])

    def o_copy(g):
        e, fc = divmod(g, _FC)
        return pltpu.make_async_copy(
            ob.at[g % _NO], o_hbm.at[:, e, :, pl.ds(fc * BFC, BFC)],
            osem.at[g % _NO])

    x_copy(0).start()
    for g in range(min(_NW, G)):
        w_copy(g).start()

    for g in range(G):
        e, fc = divmod(g, _FC)
        if fc == 0:
            x_copy(e).wait()
        w_copy(g).wait()
        if g >= _NO:
            o_copy(g - _NO).wait()
        bias_row = b_vmem[e, 0, fc * BFC:(fc + 1) * BFC]
        for bb in range(B):
            acc = jax.lax.dot_general(
                xb[e % 2, bb], wb[g % _NW],
                dimension_numbers=(((1,), (1,)), ((), ())),
                preferred_element_type=jnp.float32,
            )
            ob[g % _NO, bb] = acc + bias_row[None, :]
        o_copy(g).start()
        if g + _NW < G:
            w_copy(g + _NW).start()
        if fc == 0 and e + 1 < E:
            x_copy(e + 1).start()

    for g in range(max(G - _NO, 0), G):
        o_copy(g).wait()


@jax.jit
def kernel(x, W, b):
    B, E, N, D = x.shape
    BFC = D // _FC
    b3 = b.reshape(E, 1, D)
    return pl.pallas_call(
        _experts_kernel,
        in_specs=[
            pl.BlockSpec(memory_space=pltpu.HBM),
            pl.BlockSpec(memory_space=pltpu.HBM),
            pl.BlockSpec(memory_space=pltpu.VMEM),
        ],
        out_specs=pl.BlockSpec(memory_space=pltpu.HBM),
        out_shape=jax.ShapeDtypeStruct((B, E, N, D), x.dtype),
        scratch_shapes=[
            pltpu.VMEM((2, B, N, D), jnp.float32),
            pltpu.VMEM((_NW, BFC, D), jnp.float32),
            pltpu.VMEM((_NO, B, N, BFC), jnp.float32),
            pltpu.SemaphoreType.DMA((2,)),
            pltpu.SemaphoreType.DMA((_NW,)),
            pltpu.SemaphoreType.DMA((_NO,)),
        ],
        compiler_params=pltpu.CompilerParams(
            vmem_limit_bytes=100 * 1024 * 1024),
    )(x, W, b3)
